# Initial kernel scaffold; baseline (speedup 1.0000x reference)
import jax
import jax.numpy as jnp
from jax.experimental import pallas as pl


def kernel(class_attributes, class_idx, seen_class_embeddings, W1, b1, W2, b2, Wp, bp):
    raise NotImplementedError("write your pallas kernel here")


# speedup vs baseline: 1.6388x; 1.6388x over previous
import functools

import jax
import jax.numpy as jnp
from jax import lax
from jax.experimental import pallas as pl
from jax.experimental.pallas import tpu as pltpu
from jax.experimental.pallas import tpu_sc as plsc


@functools.lru_cache(maxsize=None)
def _build_gather(B, D, NW, NCHUNK, C):
    mesh = plsc.VectorSubcoreMesh(core_axis_name="c", subcore_axis_name="s")

    @functools.partial(
        pl.kernel,
        mesh=mesh,
        out_type=jax.ShapeDtypeStruct((B, D), jnp.float32),
        scratch_types=[
            pltpu.VMEM((NCHUNK, C), jnp.int32),
            pltpu.VMEM((C, D), jnp.float32),
            pltpu.VMEM((C, D), jnp.float32),
            pltpu.SemaphoreType.DMA,
            pltpu.SemaphoreType.DMA,
            pltpu.SemaphoreType.DMA,
            pltpu.SemaphoreType.DMA,
        ],
    )
    def k(idx_hbm, table_hbm, out_hbm, idx_v, buf0, buf1, g0, g1, s0, s1):
        wid = lax.axis_index("s") * 2 + lax.axis_index("c")
        base = wid * (NCHUNK * C)
        pltpu.sync_copy(idx_hbm.at[wid], idx_v)

        bufs = (buf0, buf1)
        gsems = (g0, g1)
        ssems = (s0, s1)
        gh = [None, None]
        sh = [None, None]
        gh[0] = pltpu.async_copy(table_hbm.at[idx_v.at[0]], bufs[0], gsems[0])
        for i in range(NCHUNK):
            cur = i % 2
            nxt = (i + 1) % 2
            if sh[nxt] is not None:
                sh[nxt].wait()
                sh[nxt] = None
            if i + 1 < NCHUNK:
                gh[nxt] = pltpu.async_copy(
                    table_hbm.at[idx_v.at[i + 1]], bufs[nxt], gsems[nxt])
            gh[cur].wait()
            sh[cur] = pltpu.async_copy(
                bufs[cur], out_hbm.at[pl.ds(base + i * C, C)], ssems[cur])
        for h in sh:
            if h is not None:
                h.wait()

    return k


def kernel(class_attributes, class_idx, seen_class_embeddings, W1, b1, W2, b2, Wp, bp):
    B = class_idx.shape[0]
    D = seen_class_embeddings.shape[1]
    NW = 32
    C = 32
    NCHUNK = B // (NW * C)
    idx = jnp.asarray(class_idx, jnp.int32).reshape(NW, NCHUNK, C)
    return _build_gather(B, D, NW, NCHUNK, C)(idx, seen_class_embeddings)


# speedup vs baseline: 1.6536x; 1.0090x over previous
import functools

import jax
import jax.numpy as jnp
from jax import lax
from jax.experimental import pallas as pl
from jax.experimental.pallas import tpu as pltpu
from jax.experimental.pallas import tpu_sc as plsc


@functools.lru_cache(maxsize=None)
def _build_gather(B, D, NW, NCHUNK, C, NBUF):
    mesh = plsc.VectorSubcoreMesh(core_axis_name="c", subcore_axis_name="s")

    scratch = [pltpu.VMEM((NCHUNK, C), jnp.int32)]
    scratch += [pltpu.VMEM((C, D), jnp.float32)] * NBUF
    scratch += [pltpu.SemaphoreType.DMA] * (2 * NBUF)

    @functools.partial(
        pl.kernel,
        mesh=mesh,
        out_type=jax.ShapeDtypeStruct((B, D), jnp.float32),
        scratch_types=scratch,
    )
    def k(idx_hbm, table_hbm, out_hbm, idx_v, *rest):
        bufs = rest[:NBUF]
        gsems = rest[NBUF:2 * NBUF]
        ssems = rest[2 * NBUF:]
        wid = lax.axis_index("s") * 2 + lax.axis_index("c")
        base = wid * (NCHUNK * C)
        pltpu.sync_copy(idx_hbm.at[wid], idx_v)

        gh = [None] * NBUF
        sh = [None] * NBUF
        for j in range(min(NBUF - 1, NCHUNK)):
            gh[j] = pltpu.async_copy(table_hbm.at[idx_v.at[j]], bufs[j], gsems[j])
        for i in range(NCHUNK):
            cur = i % NBUF
            pre = i + NBUF - 1
            if pre < NCHUNK:
                b = pre % NBUF
                if sh[b] is not None:
                    sh[b].wait()
                    sh[b] = None
                gh[b] = pltpu.async_copy(
                    table_hbm.at[idx_v.at[pre]], bufs[b], gsems[b])
            gh[cur].wait()
            sh[cur] = pltpu.async_copy(
                bufs[cur], out_hbm.at[pl.ds(base + i * C, C)], ssems[cur])
        for h in sh:
            if h is not None:
                h.wait()

    return k


def kernel(class_attributes, class_idx, seen_class_embeddings, W1, b1, W2, b2, Wp, bp):
    B = class_idx.shape[0]
    D = seen_class_embeddings.shape[1]
    NW = 32
    C = 32
    NBUF = 3
    NCHUNK = B // (NW * C)
    idx = jnp.asarray(class_idx, jnp.int32).reshape(NW, NCHUNK, C)
    return _build_gather(B, D, NW, NCHUNK, C, NBUF)(idx, seen_class_embeddings)
